# baseline (speedup 1.0000x reference)
import functools

import jax
import jax.numpy as jnp
from jax import lax
from jax.experimental import pallas as pl
from jax.experimental.pallas import tpu as pltpu
from jax.experimental.pallas import tpu_sc as plsc

B = 16384
F = 26
V = 1_040_000

NC = 2
NS = 16
NW = NC * NS
BPW = B // NW
CB = 128
NCHUNK = BPW // CB
NROWS = F * NCHUNK
FIRE = 8
NGRP = NROWS // FIRE


def _sc_lookup_sum(x_r, table, bias16):
    mesh = plsc.VectorSubcoreMesh(core_axis_name="c", subcore_axis_name="s")

    @functools.partial(
        pl.kernel,
        mesh=mesh,
        compiler_params=pltpu.CompilerParams(needs_layout_passes=False),
        out_type=jax.ShapeDtypeStruct((NW, NCHUNK, CB), jnp.float32),
        scratch_types=[
            pltpu.VMEM((NROWS * CB,), jnp.int32),
            pltpu.VMEM((NROWS * CB,), jnp.float32),
            pltpu.VMEM((NCHUNK, CB), jnp.float32),
            pltpu.VMEM((16,), jnp.float32),
            pltpu.VMEM_SHARED((V,), jnp.float32),
            pltpu.VMEM((13000,), jnp.float32),
            pltpu.VMEM((13000,), jnp.float32),
            pltpu.SemaphoreType.DMA,
            pltpu.SemaphoreType.DMA,
        ],
    )
    def k(x_hbm, table_hbm, bias_hbm, out_hbm, idx_v, vals_v, out_v, bias_v,
          table_sh, stage_a, stage_b, sem, sem2):
        sid = lax.axis_index("s")
        wid = sid * NC + lax.axis_index("c")
        vs = V // NS
        SCH = 13000
        nst = vs // SCH
        pltpu.sync_copy(bias_hbm, bias_v)
        stages = [stage_a, stage_b]
        out_cps = [None, None]
        for j in range(nst):
            off = sid * vs + j * SCH
            if out_cps[j % 2] is not None:
                out_cps[j % 2].wait()
            pltpu.async_copy(
                table_hbm.at[pl.ds(off, SCH)], stages[j % 2], sem
            ).wait()
            out_cps[j % 2] = pltpu.async_copy(
                stages[j % 2], table_sh.at[pl.ds(off, SCH)], sem2
            )
        pltpu.sync_copy(x_hbm.at[wid], idx_v)
        for cp in out_cps:
            cp.wait()
        plsc.subcore_barrier()

        pltpu.async_copy(table_sh.at[idx_v], vals_v, sem).wait()

        bv = bias_v[...]
        row16 = lax.iota(jnp.int32, 16) * F
        for g in range(BPW // 16):
            c, lb = divmod(g, CB // 16)
            acc = bv
            gbase = row16 + (g * 16 * F)
            for f in range(F):
                acc = acc + plsc.load_gather(vals_v, [gbase + f])
            out_v[c, pl.ds(lb * 16, 16)] = acc

        pltpu.sync_copy(out_v, out_hbm.at[wid])

    return k(x_r, table, bias16)


def kernel(x, fc_weight, bias):
    x_r = x.reshape(NW, BPW * F)
    table = fc_weight.reshape(V)
    bias16 = jnp.broadcast_to(bias.astype(jnp.float32), (16,))
    out = _sc_lookup_sum(x_r, table, bias16)
    return out.reshape(B, 1)


# speedup vs baseline: 1.0446x; 1.0446x over previous
import functools

import jax
import jax.numpy as jnp
from jax import lax
from jax.experimental import pallas as pl
from jax.experimental.pallas import tpu as pltpu
from jax.experimental.pallas import tpu_sc as plsc

B = 16384
F = 26
V = 1_040_000

NC = 2
NS = 16
NW = NC * NS
BPW = B // NW
CB = 128
NCHUNK = BPW // CB
NROWS = F * NCHUNK


def _sc_lookup_sum(x_r, table, bias16):
    mesh = plsc.VectorSubcoreMesh(core_axis_name="c", subcore_axis_name="s")

    @functools.partial(
        pl.kernel,
        mesh=mesh,
        compiler_params=pltpu.CompilerParams(needs_layout_passes=False),
        out_type=jax.ShapeDtypeStruct((NW, NCHUNK, CB), jnp.float32),
        scratch_types=[
            pltpu.VMEM((NROWS * CB,), jnp.int32),
            pltpu.VMEM((NROWS * CB,), jnp.float32),
            pltpu.VMEM((NCHUNK, CB), jnp.float32),
            pltpu.VMEM((16,), jnp.float32),
            pltpu.SemaphoreType.DMA,
        ],
    )
    def k(x_hbm, table_hbm, bias_hbm, out_hbm, idx_v, vals_v, out_v,
          bias_v, sem):
        wid = lax.axis_index("s") * NC + lax.axis_index("c")
        pltpu.sync_copy(bias_hbm, bias_v)
        pltpu.sync_copy(x_hbm.at[wid], idx_v)

        pltpu.async_copy(table_hbm.at[idx_v], vals_v, sem).wait()

        bv = bias_v[...]
        for c in range(NCHUNK):
            for lb in range(CB // 16):
                sl = pl.ds(lb * 16, 16)
                acc = bv
                for f in range(F):
                    acc = acc + vals_v[pl.ds((f * NCHUNK + c) * CB + lb * 16, 16)]
                out_v[c, sl] = acc

        pltpu.sync_copy(out_v, out_hbm.at[wid])

    return k(x_r, table, bias16)


def kernel(x, fc_weight, bias):
    x_r = (
        x.reshape(NW, NCHUNK, CB, F)
        .transpose(0, 3, 1, 2)
        .reshape(NW, NROWS * CB)
    )
    bias16 = jnp.broadcast_to(bias.astype(jnp.float32), (16,))
    out = _sc_lookup_sum(x_r, fc_weight.T.reshape(V), bias16)
    return out.reshape(B, 1)


# speedup vs baseline: 2.0476x; 1.9601x over previous
import functools

import jax
import jax.numpy as jnp
from jax import lax
from jax.experimental import pallas as pl
from jax.experimental.pallas import tpu as pltpu
from jax.experimental.pallas import tpu_sc as plsc

B = 16384
F = 26
V = 1_040_000

NC = 2
NS = 16
NW = NC * NS
BPW = B // NW
CB = 128
NCHUNK = BPW // CB
NROWS = F * NCHUNK


def _sc_lookup_sum(x_r, table, bias16):
    mesh = plsc.VectorSubcoreMesh(core_axis_name="c", subcore_axis_name="s")

    @functools.partial(
        pl.kernel,
        mesh=mesh,
        compiler_params=pltpu.CompilerParams(needs_layout_passes=False),
        out_type=jax.ShapeDtypeStruct((NW, NCHUNK, CB), jnp.float32),
        scratch_types=[
            pltpu.VMEM((NROWS * CB,), jnp.int32),
            pltpu.VMEM((NROWS * CB,), jnp.float32),
            pltpu.VMEM((NCHUNK, CB), jnp.float32),
            pltpu.VMEM((16,), jnp.float32),
            pltpu.SemaphoreType.DMA,
        ],
    )
    def k(x_hbm, table_hbm, bias_hbm, out_hbm, idx_v, vals_v, out_v,
          bias_v, sem):
        wid = lax.axis_index("s") * NC + lax.axis_index("c")
        pltpu.sync_copy(bias_hbm, bias_v)
        pltpu.sync_copy(x_hbm.at[wid], idx_v)

        pltpu.async_copy(table_hbm.at[0].at[idx_v], vals_v, sem).wait()

        bv = bias_v[...]
        for c in range(NCHUNK):
            for lb in range(CB // 16):
                sl = pl.ds(lb * 16, 16)
                acc = bv
                for f in range(F):
                    acc = acc + vals_v[pl.ds((f * NCHUNK + c) * CB + lb * 16, 16)]
                out_v[c, sl] = acc

        pltpu.sync_copy(out_v, out_hbm.at[wid])

    return k(x_r, table, bias16)


def kernel(x, fc_weight, bias):
    x_r = (
        x.reshape(NW, NCHUNK, CB, F)
        .transpose(0, 3, 1, 2)
        .reshape(NW, NROWS * CB)
    )
    bias16 = jnp.broadcast_to(bias.astype(jnp.float32), (16,))
    out = _sc_lookup_sum(x_r, fc_weight.T, bias16)
    return out.reshape(B, 1)


# speedup vs baseline: 2.3732x; 1.1590x over previous
import functools

import jax
import jax.numpy as jnp
from jax import lax
from jax.experimental import pallas as pl
from jax.experimental.pallas import tpu as pltpu
from jax.experimental.pallas import tpu_sc as plsc

B = 16384
F = 26
V = 1_040_000

NC = 2
NS = 16
NW = NC * NS
BPW = B // NW
NIDX = BPW * F
VS = V // NS
SCH = 13000


def _sc_lookup_sum(xT, tableT, bias):
    mesh = plsc.VectorSubcoreMesh(core_axis_name="c", subcore_axis_name="s")

    @functools.partial(
        pl.kernel,
        mesh=mesh,
        compiler_params=pltpu.CompilerParams(needs_layout_passes=False),
        out_type=jax.ShapeDtypeStruct((NW, BPW), jnp.float32),
        scratch_types=[
            pltpu.VMEM((NIDX,), jnp.int32),
            pltpu.VMEM((NIDX,), jnp.float32),
            pltpu.VMEM((BPW,), jnp.float32),
            pltpu.VMEM((16,), jnp.float32),
            pltpu.VMEM((SCH,), jnp.float32),
            pltpu.VMEM((SCH,), jnp.float32),
            pltpu.VMEM_SHARED((V,), jnp.float32),
            pltpu.SemaphoreType.DMA,
            pltpu.SemaphoreType.DMA,
            pltpu.SemaphoreType.DMA,
        ],
    )
    def k(xT_hbm, table_hbm, bias_hbm, out_hbm, idx_v, vals_v, out_v, bias_v,
          stage_a, stage_b, table_sh, sem, sem2, sem3):
        sid = lax.axis_index("s")
        wid = sid * NC + lax.axis_index("c")
        base = wid * BPW

        pltpu.sync_copy(bias_hbm, bias_v)
        idx_cps = [
            pltpu.async_copy(
                xT_hbm.at[f, pl.ds(base, BPW)],
                idx_v.at[pl.ds(f * BPW, BPW)], sem3,
            )
            for f in range(F)
        ]

        stages = [stage_a, stage_b]
        out_cps = [None, None]
        for j in range(VS // SCH):
            off = sid * VS + j * SCH
            if out_cps[j % 2] is not None:
                out_cps[j % 2].wait()
            pltpu.async_copy(
                table_hbm.at[0].at[pl.ds(off, SCH)], stages[j % 2], sem
            ).wait()
            out_cps[j % 2] = pltpu.async_copy(
                stages[j % 2], table_sh.at[pl.ds(off, SCH)], sem2
            )
        for cp in out_cps:
            cp.wait()
        for cp in idx_cps:
            cp.wait()
        plsc.subcore_barrier()

        pltpu.async_copy(table_sh.at[idx_v], vals_v, sem).wait()

        bv = bias_v[...]
        for g in range(BPW // 16):
            acc = bv
            for f in range(F):
                acc = acc + vals_v[pl.ds(f * BPW + g * 16, 16)]
            out_v[pl.ds(g * 16, 16)] = acc

        pltpu.sync_copy(out_v, out_hbm.at[wid])

    return k(xT, tableT, bias)


def kernel(x, fc_weight, bias):
    bias16 = jnp.broadcast_to(bias.astype(jnp.float32), (16,))
    out = _sc_lookup_sum(x.T, fc_weight.T, bias16)
    return out.reshape(B, 1)


# speedup vs baseline: 2.3975x; 1.0102x over previous
import functools

import jax
import jax.numpy as jnp
from jax import lax
from jax.experimental import pallas as pl
from jax.experimental.pallas import tpu as pltpu
from jax.experimental.pallas import tpu_sc as plsc

B = 16384
F = 26
V = 1_040_000

NC = 2
NS = 16
NW = NC * NS
BPW = B // NW
NIDX = BPW * F
VS = V // NS
SCH = 13000


def _sc_lookup_sum(xT, tableT, bias):
    mesh = plsc.VectorSubcoreMesh(core_axis_name="c", subcore_axis_name="s")

    @functools.partial(
        pl.kernel,
        mesh=mesh,
        compiler_params=pltpu.CompilerParams(needs_layout_passes=False),
        out_type=jax.ShapeDtypeStruct((1, B), jnp.float32),
        scratch_types=[
            pltpu.VMEM((NIDX,), jnp.int32),
            pltpu.VMEM((NIDX,), jnp.float32),
            pltpu.VMEM((BPW,), jnp.float32),
            pltpu.VMEM((16,), jnp.float32),
            pltpu.VMEM((SCH,), jnp.float32),
            pltpu.VMEM((SCH,), jnp.float32),
            pltpu.VMEM_SHARED((V,), jnp.float32),
            pltpu.SemaphoreType.DMA,
            pltpu.SemaphoreType.DMA,
            pltpu.SemaphoreType.DMA,
        ],
    )
    def k(xT_hbm, table_hbm, bias_hbm, out_hbm, idx_v, vals_v, out_v, bias_v,
          stage_a, stage_b, table_sh, sem, sem2, sem3):
        sid = lax.axis_index("s")
        wid = sid * NC + lax.axis_index("c")
        base = wid * BPW

        zidx = lax.iota(jnp.int32, 16) * 0
        pltpu.async_copy(bias_hbm.at[zidx], bias_v, sem).wait()
        idx_cps = [
            pltpu.async_copy(
                xT_hbm.at[f, pl.ds(base, BPW)],
                idx_v.at[pl.ds(f * BPW, BPW)], sem3,
            )
            for f in range(F)
        ]

        stages = [stage_a, stage_b]
        out_cps = [None, None]
        for j in range(VS // SCH):
            off = sid * VS + j * SCH
            if out_cps[j % 2] is not None:
                out_cps[j % 2].wait()
            pltpu.async_copy(
                table_hbm.at[0].at[pl.ds(off, SCH)], stages[j % 2], sem
            ).wait()
            out_cps[j % 2] = pltpu.async_copy(
                stages[j % 2], table_sh.at[pl.ds(off, SCH)], sem2
            )
        for cp in out_cps:
            cp.wait()
        for cp in idx_cps:
            cp.wait()
        plsc.subcore_barrier()

        pltpu.async_copy(table_sh.at[idx_v], vals_v, sem).wait()

        bv = bias_v[...]
        for g in range(BPW // 16):
            acc = bv
            for f in range(F):
                acc = acc + vals_v[pl.ds(f * BPW + g * 16, 16)]
            out_v[pl.ds(g * 16, 16)] = acc

        pltpu.sync_copy(out_v, out_hbm.at[0].at[pl.ds(base, BPW)])

    return k(xT, tableT, bias)


def kernel(x, fc_weight, bias):
    out = _sc_lookup_sum(x.T, fc_weight.T, bias.astype(jnp.float32))
    return out.T


# speedup vs baseline: 2.4110x; 1.0056x over previous
import functools

import jax
import jax.numpy as jnp
from jax import lax
from jax.experimental import pallas as pl
from jax.experimental.pallas import tpu as pltpu
from jax.experimental.pallas import tpu_sc as plsc

B = 16384
F = 26
V = 1_040_000

NC = 2
NS = 16
NW = NC * NS
BPW = B // NW
NIDX = BPW * F
VS = V // NS
SCH = 13000


def _sc_lookup_sum(xT, tableT, bias):
    mesh = plsc.VectorSubcoreMesh(core_axis_name="c", subcore_axis_name="s")

    @functools.partial(
        pl.kernel,
        mesh=mesh,
        compiler_params=pltpu.CompilerParams(needs_layout_passes=False),
        out_type=jax.ShapeDtypeStruct((1, B), jnp.float32),
        scratch_types=[
            pltpu.VMEM((NIDX,), jnp.int32),
            pltpu.VMEM((NIDX,), jnp.float32),
            pltpu.VMEM((BPW,), jnp.float32),
            pltpu.VMEM((16,), jnp.float32),
            pltpu.VMEM((SCH,), jnp.float32),
            pltpu.VMEM((SCH,), jnp.float32),
            pltpu.VMEM_SHARED((V,), jnp.float32),
            pltpu.SemaphoreType.DMA,
            pltpu.SemaphoreType.DMA,
            pltpu.SemaphoreType.DMA,
        ],
    )
    def k(xT_hbm, table_hbm, bias_hbm, out_hbm, idx_v, vals_v, out_v, bias_v,
          stage_a, stage_b, table_sh, sem, sem2, sem3):
        sid = lax.axis_index("s")
        wid = sid * NC + lax.axis_index("c")
        base = wid * BPW

        zidx = lax.iota(jnp.int32, 16) * 0
        bias_cp = pltpu.async_copy(bias_hbm.at[zidx], bias_v, sem3)
        idx_cps = [
            pltpu.async_copy(
                xT_hbm.at[f, pl.ds(base, BPW)],
                idx_v.at[pl.ds(f * BPW, BPW)], sem3,
            )
            for f in range(F)
        ]

        stages = [stage_a, stage_b]
        out_cps = [None, None]
        for j in range(VS // SCH):
            off = sid * VS + j * SCH
            if out_cps[j % 2] is not None:
                out_cps[j % 2].wait()
            pltpu.async_copy(
                table_hbm.at[0].at[pl.ds(off, SCH)], stages[j % 2], sem
            ).wait()
            out_cps[j % 2] = pltpu.async_copy(
                stages[j % 2], table_sh.at[pl.ds(off, SCH)], sem2
            )
        for cp in out_cps:
            cp.wait()
        for cp in idx_cps:
            cp.wait()
        plsc.subcore_barrier()

        HF = F // 2
        g1 = pltpu.async_copy(
            table_sh.at[idx_v.at[pl.ds(0, HF * BPW)]],
            vals_v.at[pl.ds(0, HF * BPW)], sem,
        )
        g2 = pltpu.async_copy(
            table_sh.at[idx_v.at[pl.ds(HF * BPW, (F - HF) * BPW)]],
            vals_v.at[pl.ds(HF * BPW, (F - HF) * BPW)], sem2,
        )
        bias_cp.wait()
        g1.wait()
        bv = bias_v[...]
        accs = []
        for g in range(BPW // 16):
            acc = bv
            for f in range(HF):
                acc = acc + vals_v[pl.ds(f * BPW + g * 16, 16)]
            accs.append(acc)
        g2.wait()
        for g in range(BPW // 16):
            acc = accs[g]
            for f in range(HF, F):
                acc = acc + vals_v[pl.ds(f * BPW + g * 16, 16)]
            out_v[pl.ds(g * 16, 16)] = acc

        pltpu.sync_copy(out_v, out_hbm.at[0].at[pl.ds(base, BPW)])

    return k(xT, tableT, bias)


def kernel(x, fc_weight, bias):
    out = _sc_lookup_sum(x.T, fc_weight.T, bias.astype(jnp.float32))
    return out.T
